# Initial kernel scaffold; baseline (speedup 1.0000x reference)
import jax
import jax.numpy as jnp
from jax.experimental import pallas as pl


def kernel(x, edge_index, params):
    raise NotImplementedError("write your pallas kernel here")


# speedup vs baseline: 4.9287x; 4.9287x over previous
import functools

import jax
import jax.numpy as jnp
from jax import lax
from jax.experimental import pallas as pl
from jax.experimental.pallas import tpu as pltpu
from jax.experimental.pallas import tpu_sc as plsc

N = 50000
E = 800000
NC = 2
NS = 16
LANES = 128
R = 6400
CH = 40
NA = 50048
RPT = NA // NS

CHK = 5000
NB = N // CHK


def _segsum_partials_16(src2d, dst2d, x, zeros):
    mesh = plsc.VectorSubcoreMesh(core_axis_name="c", subcore_axis_name="s")
    rpw = R // (NC * NS)

    @functools.partial(
        pl.kernel,
        out_type=jax.ShapeDtypeStruct((NC, NA, 16), jnp.float32),
        mesh=mesh,
        compiler_params=pltpu.CompilerParams(use_tc_tiling_on_sc=False),
        scratch_types=[
            pltpu.VMEM((CH, LANES), jnp.int32),
            pltpu.VMEM((CH, LANES), jnp.int32),
            pltpu.VMEM((LANES, 16), jnp.float32),
            pltpu.VMEM_SHARED((NA, 16), jnp.float32),
            pltpu.SemaphoreType.DMA,
        ],
    )
    def k(src_hbm, dst_hbm, x_hbm, z_hbm, out_hbm, src_v, dst_v, rows_v, acc, sem):
        c = lax.axis_index("c")
        s = lax.axis_index("s")
        tbase = s * RPT
        pltpu.sync_copy(z_hbm, acc.at[pl.ds(tbase, RPT)])
        plsc.subcore_barrier()
        base = (s * NC + c) * rpw

        @pl.loop(0, rpw // CH)
        def _(g):
            pltpu.sync_copy(src_hbm.at[pl.ds(base + g * CH, CH)], src_v)
            pltpu.sync_copy(dst_hbm.at[pl.ds(base + g * CH, CH)], dst_v)

            @pl.loop(0, CH)
            def _(j):
                pltpu.async_copy(x_hbm.at[src_v.at[j]], rows_v, sem).wait()
                pltpu.sync_copy(rows_v, acc.at[dst_v.at[j]], add=True)

        plsc.subcore_barrier()
        pltpu.sync_copy(acc.at[pl.ds(tbase, RPT)],
                        out_hbm.at[c, pl.ds(tbase, RPT)])

    return k(src2d, dst2d, x, zeros)


def _segsum_cols_32(src2d, dst2d, h0, h1, zeros):
    mesh = plsc.VectorSubcoreMesh(core_axis_name="c", subcore_axis_name="s")
    rpt_idx = R // NS

    @functools.partial(
        pl.kernel,
        out_type=jax.ShapeDtypeStruct((NC, NA, 32), jnp.float32),
        mesh=mesh,
        compiler_params=pltpu.CompilerParams(use_tc_tiling_on_sc=False),
        scratch_types=[
            pltpu.VMEM((CH, LANES), jnp.int32),
            pltpu.VMEM((CH, LANES), jnp.int32),
            pltpu.VMEM((LANES, 32), jnp.float32),
            pltpu.VMEM_SHARED((NA, 32), jnp.float32),
            pltpu.SemaphoreType.DMA,
        ],
    )
    def k(src_hbm, dst_hbm, h0_hbm, h1_hbm, z_hbm, out_hbm,
          src_v, dst_v, rows_v, acc, sem):
        c = lax.axis_index("c")
        s = lax.axis_index("s")
        tbase = s * RPT
        pltpu.sync_copy(z_hbm, acc.at[pl.ds(tbase, RPT)])
        plsc.subcore_barrier()
        base = s * rpt_idx

        @pl.loop(0, rpt_idx // CH)
        def _(g):
            pltpu.sync_copy(src_hbm.at[pl.ds(base + g * CH, CH)], src_v)
            pltpu.sync_copy(dst_hbm.at[pl.ds(base + g * CH, CH)], dst_v)

            @pl.loop(0, CH)
            def _(j):
                @pl.when(c == 0)
                def _():
                    pltpu.async_copy(h0_hbm.at[src_v.at[j]], rows_v, sem).wait()

                @pl.when(c == 1)
                def _():
                    pltpu.async_copy(h1_hbm.at[src_v.at[j]], rows_v, sem).wait()

                pltpu.sync_copy(rows_v, acc.at[dst_v.at[j]], add=True)

        plsc.subcore_barrier()
        pltpu.sync_copy(acc.at[pl.ds(tbase, RPT)],
                        out_hbm.at[c, pl.ds(tbase, RPT)])

    return k(src2d, dst2d, h0, h1, zeros)


def _gram(a, b):
    return lax.dot_general(a, b, (((0,), (0,)), ((), ())),
                           preferred_element_type=jnp.float32,
                           precision=lax.Precision.HIGHEST)


def _dot_hi(a, b):
    return jnp.dot(a, b, preferred_element_type=jnp.float32,
                   precision=lax.Precision.HIGHEST)


def _dot(a, b):
    return jnp.dot(a, b, preferred_element_type=jnp.float32)


def _acc(ref, val, i):
    @pl.when(i == 0)
    def _():
        ref[...] = val

    @pl.when(i > 0)
    def _():
        ref[...] += val


def _chunk_spec(width):
    return pl.BlockSpec((CHK, width), lambda i: (i, 0))


def _pair_spec(width):
    return pl.BlockSpec((2, CHK, width), lambda i: (0, i, 0))


def _full_spec(shape):
    nd = len(shape)
    return pl.BlockSpec(shape, lambda i: (0,) * nd)


def _stats0(x, agg0p, eps):
    def body(x_ref, a_ref, eps_ref, su_ref, gram_ref, sx_ref, pooled_ref):
        i = pl.program_id(0)
        p = ((1.0 + eps_ref[0, 0]) * x_ref[...]
             + a_ref[0, :, :] + a_ref[1, :, :])
        pooled_ref[...] = p
        _acc(su_ref, jnp.sum(p, axis=0, keepdims=True), i)
        _acc(gram_ref, _gram(p, p), i)
        _acc(sx_ref, jnp.sum(x_ref[...], axis=0, keepdims=True), i)

    return pl.pallas_call(
        body,
        grid=(NB,),
        in_specs=[_chunk_spec(16), _pair_spec(16), _full_spec((1, 1))],
        out_specs=(_full_spec((1, 16)), _full_spec((16, 16)),
                   _full_spec((1, 16)), _chunk_spec(16)),
        out_shape=(jax.ShapeDtypeStruct((1, 16), jnp.float32),
                   jax.ShapeDtypeStruct((16, 16), jnp.float32),
                   jax.ShapeDtypeStruct((1, 16), jnp.float32),
                   jax.ShapeDtypeStruct((N, 16), jnp.float32)),
    )(x, agg0p, eps)


def _transform(pooled, su, gram, W1, b1, g1, be1, width_in):
    def body(p_ref, su_ref, gram_ref, W1_ref, b1_ref, g1_ref, be1_ref,
             r1_ref, su2_ref, gram2_ref):
        i = pl.program_id(0)
        W1v = W1_ref[...]
        mw = _dot_hi(su_ref[...] / N, W1v)
        q = jnp.sum(W1v * _dot_hi(gram_ref[...] / N, W1v), axis=0,
                    keepdims=True)
        var = q - mw * mw
        a1 = g1_ref[...] * lax.rsqrt(var + 1e-5)
        c1 = be1_ref[...] - mw * a1
        r = jnp.maximum(_dot(p_ref[...], W1v * a1) + c1, 0.0)
        r1_ref[...] = r
        _acc(su2_ref, jnp.sum(r, axis=0, keepdims=True), i)
        _acc(gram2_ref, _gram(r, r), i)

    return pl.pallas_call(
        body,
        grid=(NB,),
        in_specs=[_chunk_spec(width_in), _full_spec((1, width_in)),
                  _full_spec((width_in, width_in)),
                  _full_spec((width_in, 64)),
                  _full_spec((1, 64)), _full_spec((1, 64)),
                  _full_spec((1, 64))],
        out_specs=(_chunk_spec(64), _full_spec((1, 64)),
                   _full_spec((64, 64))),
        out_shape=(jax.ShapeDtypeStruct((N, 64), jnp.float32),
                   jax.ShapeDtypeStruct((1, 64), jnp.float32),
                   jax.ShapeDtypeStruct((64, 64), jnp.float32)),
    )(pooled, su, gram, W1, b1, g1, be1)


def _emit_h1(r1, su2, gram2, W2, b2, g2, be2):
    def body(r_ref, su_ref, gram_ref, W2_ref, b2_ref, g2_ref, be2_ref,
             out_ref):
        W2v = W2_ref[...]
        mw = _dot_hi(su_ref[...] / N, W2v)
        q = jnp.sum(W2v * _dot_hi(gram_ref[...] / N, W2v), axis=0,
                    keepdims=True)
        var = q - mw * mw
        a2 = g2_ref[...] * lax.rsqrt(var + 1e-5)
        c2 = be2_ref[...] - mw * a2
        W2s = W2v * a2
        r = r_ref[...]
        out_ref[0, :, :] = jnp.maximum(_dot(r, W2s[:, :32]) + c2[:, :32], 0.0)
        out_ref[1, :, :] = jnp.maximum(_dot(r, W2s[:, 32:]) + c2[:, 32:], 0.0)

    return pl.pallas_call(
        body,
        grid=(NB,),
        in_specs=[_chunk_spec(64), _full_spec((1, 64)), _full_spec((64, 64)),
                  _full_spec((64, 64)), _full_spec((1, 64)),
                  _full_spec((1, 64)), _full_spec((1, 64))],
        out_specs=_pair_spec(32),
        out_shape=jax.ShapeDtypeStruct((2, N, 32), jnp.float32),
    )(r1, su2, gram2, W2, b2, g2, be2)


def _stats1(h1h, agg1p, eps):
    def body(h_ref, a_ref, eps_ref, p_ref, su_ref, gram_ref, s1_ref):
        i = pl.program_id(0)
        ep = 1.0 + eps_ref[0, 0]
        pa = ep * h_ref[0, :, :] + a_ref[0, :, :]
        pb = ep * h_ref[1, :, :] + a_ref[1, :, :]
        p = jnp.concatenate([pa, pb], axis=1)
        p_ref[...] = p
        _acc(su_ref, jnp.sum(p, axis=0, keepdims=True), i)
        _acc(gram_ref, _gram(p, p), i)
        s1 = jnp.concatenate(
            [jnp.sum(h_ref[0, :, :], axis=0, keepdims=True),
             jnp.sum(h_ref[1, :, :], axis=0, keepdims=True)], axis=1)
        _acc(s1_ref, s1, i)

    return pl.pallas_call(
        body,
        grid=(NB,),
        in_specs=[_pair_spec(32), _pair_spec(32), _full_spec((1, 1))],
        out_specs=(_chunk_spec(64), _full_spec((1, 64)),
                   _full_spec((64, 64)), _full_spec((1, 64))),
        out_shape=(jax.ShapeDtypeStruct((N, 64), jnp.float32),
                   jax.ShapeDtypeStruct((1, 64), jnp.float32),
                   jax.ShapeDtypeStruct((64, 64), jnp.float32),
                   jax.ShapeDtypeStruct((1, 64), jnp.float32)),
    )(h1h, agg1p, eps)


def _readout(r1, su2, gram2, W2, b2, g2, be2, sx, s1,
             P0, P1, P2, pb, M0, mb0, M1, mb1, Wm, bm, Wv, bv):
    def body(r_ref, su_ref, gram_ref, W2_ref, b2_ref, g2_ref, be2_ref,
             sx_ref, s1_ref, P0_ref, P1_ref, P2_ref, pb_ref,
             M0_ref, mb0_ref, M1_ref, mb1_ref, Wm_ref, bm_ref, Wv_ref, bv_ref,
             mean_ref, logvar_ref, s2_ref):
        i = pl.program_id(0)
        W2v = W2_ref[...]
        mw = _dot_hi(su_ref[...] / N, W2v)
        q = jnp.sum(W2v * _dot_hi(gram_ref[...] / N, W2v), axis=0,
                    keepdims=True)
        var = q - mw * mw
        a2 = g2_ref[...] * lax.rsqrt(var + 1e-5)
        c2 = be2_ref[...] - mw * a2
        h2 = jnp.maximum(_dot(r_ref[...], W2v * a2) + c2, 0.0)
        _acc(s2_ref, jnp.sum(h2, axis=0, keepdims=True), i)

        @pl.when(i == NB - 1)
        def _():
            s1 = s1_ref[...]
            score = (_dot(sx_ref[...], P0_ref[...])
                     + _dot(s1[:, :32], P1_ref[:32, :])
                     + _dot(s1[:, 32:], P1_ref[32:, :])
                     + _dot(s2_ref[...], P2_ref[...]) + pb_ref[...])
            f = jnp.maximum(_dot(score, M0_ref[...]) + mb0_ref[...], 0.0)
            f = jnp.maximum(_dot(f, M1_ref[...]) + mb1_ref[...], 0.0)
            mean_ref[...] = _dot(f, Wm_ref[...]) + bm_ref[...]
            logvar_ref[...] = _dot(f, Wv_ref[...]) + bv_ref[...]

    return pl.pallas_call(
        body,
        grid=(NB,),
        in_specs=[_chunk_spec(64), _full_spec((1, 64)), _full_spec((64, 64)),
                  _full_spec((64, 64)), _full_spec((1, 64)),
                  _full_spec((1, 64)), _full_spec((1, 64)),
                  _full_spec((1, 16)), _full_spec((1, 64)),
                  _full_spec((16, 256)), _full_spec((64, 256)),
                  _full_spec((64, 256)), _full_spec((1, 256)),
                  _full_spec((256, 256)), _full_spec((1, 256)),
                  _full_spec((256, 128)), _full_spec((1, 128)),
                  _full_spec((128, 64)), _full_spec((1, 64)),
                  _full_spec((128, 64)), _full_spec((1, 64))],
        out_specs=(_full_spec((1, 64)), _full_spec((1, 64)),
                   _full_spec((1, 64))),
        out_shape=(jax.ShapeDtypeStruct((1, 64), jnp.float32),
                   jax.ShapeDtypeStruct((1, 64), jnp.float32),
                   jax.ShapeDtypeStruct((1, 64), jnp.float32)),
    )(r1, su2, gram2, W2, b2, g2, be2, sx, s1,
      P0, P1, P2, pb, M0, mb0, M1, mb1, Wm, bm, Wv, bv)


def kernel(x, edge_index, params):
    src = edge_index[0].astype(jnp.int32)
    dst = edge_index[1].astype(jnp.int32)
    pad = R * LANES - E
    src2d = jnp.concatenate(
        [src, jnp.zeros((pad,), jnp.int32)]).reshape(R, LANES)
    trash = N + (jnp.arange(pad, dtype=jnp.int32) % (NA - N))
    dst2d = jnp.concatenate([dst, trash]).reshape(R, LANES)

    z16 = jnp.zeros((RPT, 16), jnp.float32)
    z32 = jnp.zeros((RPT, 32), jnp.float32)

    def v(name):
        return params[name].reshape(1, -1)

    eps0 = params['eps_0'].reshape(1, 1)
    eps1 = params['eps_1'].reshape(1, 1)

    agg0p = _segsum_partials_16(src2d, dst2d, x, z16)
    su, gram, sx, pooled = _stats0(x, agg0p, eps0)
    r1, su2, gram2 = _transform(pooled, su, gram, params['W1_0'], v('b1_0'),
                                v('g1_0'), v('be1_0'), 16)
    h1h = _emit_h1(r1, su2, gram2,
                   params['W2_0'], v('b2_0'), v('g_0'), v('be_0'))

    agg1p = _segsum_cols_32(src2d, dst2d, h1h[0], h1h[1], z32)
    pooled1, su1, gram1, s1 = _stats1(h1h, agg1p, eps1)
    r1b, su2b, gram2b = _transform(pooled1, su1, gram1,
                                   params['W1_1'], v('b1_1'), v('g1_1'),
                                   v('be1_1'), 64)
    pb = (params['pb_0'] + params['pb_1'] + params['pb_2']).reshape(1, -1)
    mean, logvar, _ = _readout(
        r1b, su2b, gram2b, params['W2_1'], v('b2_1'), v('g_1'), v('be_1'),
        sx, s1, params['P_0'], params['P_1'], params['P_2'], pb,
        params['M_0'], v('mb_0'), params['M_1'], v('mb_1'),
        params['Wm'], v('bm'), params['Wv'], v('bv'))
    return (mean, logvar)


# speedup vs baseline: 6.0264x; 1.2227x over previous
import functools

import jax
import jax.numpy as jnp
from jax import lax
from jax.experimental import pallas as pl
from jax.experimental.pallas import tpu as pltpu
from jax.experimental.pallas import tpu_sc as plsc

N = 50000
E = 800000
NC = 2
NS = 16
LANES = 128
R = 6400
CH = 40
K = 8
K32 = 4
NA = 50048
RPT = NA // NS

CHK = 5000
NB = N // CHK


def _segsum_partials_16(src2d, dst2d, x, zeros):
    mesh = plsc.VectorSubcoreMesh(core_axis_name="c", subcore_axis_name="s")
    rpw = R // (NC * NS)

    @functools.partial(
        pl.kernel,
        out_type=jax.ShapeDtypeStruct((NC, NA, 16), jnp.float32),
        mesh=mesh,
        compiler_params=pltpu.CompilerParams(use_tc_tiling_on_sc=False),
        scratch_types=[
            pltpu.VMEM((CH, LANES), jnp.int32),
            pltpu.VMEM((CH, LANES), jnp.int32),
            pltpu.VMEM((K, LANES, 16), jnp.float32),
            pltpu.VMEM_SHARED((NA, 16), jnp.float32),
            pltpu.SemaphoreType.DMA,
            pltpu.SemaphoreType.DMA,
        ],
    )
    def k(src_hbm, dst_hbm, x_hbm, z_hbm, out_hbm, src_v, dst_v, rows_v, acc,
          gsem, ssem):
        c = lax.axis_index("c")
        s = lax.axis_index("s")
        tbase = s * RPT
        pltpu.sync_copy(z_hbm, acc.at[pl.ds(tbase, RPT)])
        plsc.subcore_barrier()
        base = (s * NC + c) * rpw

        @pl.loop(0, rpw // CH)
        def _(g):
            pltpu.sync_copy(src_hbm.at[pl.ds(base + g * CH, CH)], src_v)
            pltpu.sync_copy(dst_hbm.at[pl.ds(base + g * CH, CH)], dst_v)

            @pl.loop(0, CH // K)
            def _(t):
                hs = [pltpu.async_copy(x_hbm.at[src_v.at[t * K + b]],
                                       rows_v.at[b], gsem)
                      for b in range(K)]
                for h in hs:
                    h.wait()
                ss = [pltpu.async_copy(rows_v.at[b],
                                       acc.at[dst_v.at[t * K + b]], ssem,
                                       add=True)
                      for b in range(K)]
                for h in ss:
                    h.wait()

        plsc.subcore_barrier()
        pltpu.sync_copy(acc.at[pl.ds(tbase, RPT)],
                        out_hbm.at[c, pl.ds(tbase, RPT)])

    return k(src2d, dst2d, x, zeros)


def _segsum_cols_32(src2d, dst2d, h0, h1, zeros):
    mesh = plsc.VectorSubcoreMesh(core_axis_name="c", subcore_axis_name="s")
    rpt_idx = R // NS

    @functools.partial(
        pl.kernel,
        out_type=jax.ShapeDtypeStruct((NC, NA, 32), jnp.float32),
        mesh=mesh,
        compiler_params=pltpu.CompilerParams(use_tc_tiling_on_sc=False),
        scratch_types=[
            pltpu.VMEM((CH, LANES), jnp.int32),
            pltpu.VMEM((CH, LANES), jnp.int32),
            pltpu.VMEM((K32, LANES, 32), jnp.float32),
            pltpu.VMEM_SHARED((NA, 32), jnp.float32),
            pltpu.SemaphoreType.DMA,
            pltpu.SemaphoreType.DMA,
        ],
    )
    def k(src_hbm, dst_hbm, h0_hbm, h1_hbm, z_hbm, out_hbm,
          src_v, dst_v, rows_v, acc, gsem, ssem):
        c = lax.axis_index("c")
        s = lax.axis_index("s")
        tbase = s * RPT
        pltpu.sync_copy(z_hbm, acc.at[pl.ds(tbase, RPT)])
        plsc.subcore_barrier()
        base = s * rpt_idx

        @pl.loop(0, rpt_idx // CH)
        def _(g):
            pltpu.sync_copy(src_hbm.at[pl.ds(base + g * CH, CH)], src_v)
            pltpu.sync_copy(dst_hbm.at[pl.ds(base + g * CH, CH)], dst_v)

            @pl.loop(0, CH // K32)
            def _(t):
                @pl.when(c == 0)
                def _():
                    hs = [pltpu.async_copy(h0_hbm.at[src_v.at[t * K32 + b]],
                                           rows_v.at[b], gsem)
                          for b in range(K32)]
                    for h in hs:
                        h.wait()

                @pl.when(c == 1)
                def _():
                    hs = [pltpu.async_copy(h1_hbm.at[src_v.at[t * K32 + b]],
                                           rows_v.at[b], gsem)
                          for b in range(K32)]
                    for h in hs:
                        h.wait()

                ss = [pltpu.async_copy(rows_v.at[b],
                                       acc.at[dst_v.at[t * K32 + b]], ssem,
                                       add=True)
                      for b in range(K32)]
                for h in ss:
                    h.wait()

        plsc.subcore_barrier()
        pltpu.sync_copy(acc.at[pl.ds(tbase, RPT)],
                        out_hbm.at[c, pl.ds(tbase, RPT)])

    return k(src2d, dst2d, h0, h1, zeros)


def _gram(a, b):
    return lax.dot_general(a, b, (((0,), (0,)), ((), ())),
                           preferred_element_type=jnp.float32,
                           precision=lax.Precision.HIGHEST)


def _dot_hi(a, b):
    return jnp.dot(a, b, preferred_element_type=jnp.float32,
                   precision=lax.Precision.HIGHEST)


def _dot(a, b):
    return jnp.dot(a, b, preferred_element_type=jnp.float32)


def _acc(ref, val, i):
    @pl.when(i == 0)
    def _():
        ref[...] = val

    @pl.when(i > 0)
    def _():
        ref[...] += val


def _chunk_spec(width):
    return pl.BlockSpec((CHK, width), lambda i: (i, 0))


def _pair_spec(width):
    return pl.BlockSpec((2, CHK, width), lambda i: (0, i, 0))


def _full_spec(shape):
    nd = len(shape)
    return pl.BlockSpec(shape, lambda i: (0,) * nd)


def _stats0(x, agg0p, eps):
    def body(x_ref, a_ref, eps_ref, su_ref, gram_ref, sx_ref, pooled_ref):
        i = pl.program_id(0)
        p = ((1.0 + eps_ref[0, 0]) * x_ref[...]
             + a_ref[0, :, :] + a_ref[1, :, :])
        pooled_ref[...] = p
        _acc(su_ref, jnp.sum(p, axis=0, keepdims=True), i)
        _acc(gram_ref, _gram(p, p), i)
        _acc(sx_ref, jnp.sum(x_ref[...], axis=0, keepdims=True), i)

    return pl.pallas_call(
        body,
        grid=(NB,),
        in_specs=[_chunk_spec(16), _pair_spec(16), _full_spec((1, 1))],
        out_specs=(_full_spec((1, 16)), _full_spec((16, 16)),
                   _full_spec((1, 16)), _chunk_spec(16)),
        out_shape=(jax.ShapeDtypeStruct((1, 16), jnp.float32),
                   jax.ShapeDtypeStruct((16, 16), jnp.float32),
                   jax.ShapeDtypeStruct((1, 16), jnp.float32),
                   jax.ShapeDtypeStruct((N, 16), jnp.float32)),
    )(x, agg0p, eps)


def _transform(pooled, su, gram, W1, b1, g1, be1, width_in):
    def body(p_ref, su_ref, gram_ref, W1_ref, b1_ref, g1_ref, be1_ref,
             r1_ref, su2_ref, gram2_ref):
        i = pl.program_id(0)
        W1v = W1_ref[...]
        mw = _dot_hi(su_ref[...] / N, W1v)
        q = jnp.sum(W1v * _dot_hi(gram_ref[...] / N, W1v), axis=0,
                    keepdims=True)
        var = q - mw * mw
        a1 = g1_ref[...] * lax.rsqrt(var + 1e-5)
        c1 = be1_ref[...] - mw * a1
        r = jnp.maximum(_dot(p_ref[...], W1v * a1) + c1, 0.0)
        r1_ref[...] = r
        _acc(su2_ref, jnp.sum(r, axis=0, keepdims=True), i)
        _acc(gram2_ref, _gram(r, r), i)

    return pl.pallas_call(
        body,
        grid=(NB,),
        in_specs=[_chunk_spec(width_in), _full_spec((1, width_in)),
                  _full_spec((width_in, width_in)),
                  _full_spec((width_in, 64)),
                  _full_spec((1, 64)), _full_spec((1, 64)),
                  _full_spec((1, 64))],
        out_specs=(_chunk_spec(64), _full_spec((1, 64)),
                   _full_spec((64, 64))),
        out_shape=(jax.ShapeDtypeStruct((N, 64), jnp.float32),
                   jax.ShapeDtypeStruct((1, 64), jnp.float32),
                   jax.ShapeDtypeStruct((64, 64), jnp.float32)),
    )(pooled, su, gram, W1, b1, g1, be1)


def _emit_h1(r1, su2, gram2, W2, b2, g2, be2):
    def body(r_ref, su_ref, gram_ref, W2_ref, b2_ref, g2_ref, be2_ref,
             out_ref):
        W2v = W2_ref[...]
        mw = _dot_hi(su_ref[...] / N, W2v)
        q = jnp.sum(W2v * _dot_hi(gram_ref[...] / N, W2v), axis=0,
                    keepdims=True)
        var = q - mw * mw
        a2 = g2_ref[...] * lax.rsqrt(var + 1e-5)
        c2 = be2_ref[...] - mw * a2
        W2s = W2v * a2
        r = r_ref[...]
        out_ref[0, :, :] = jnp.maximum(_dot(r, W2s[:, :32]) + c2[:, :32], 0.0)
        out_ref[1, :, :] = jnp.maximum(_dot(r, W2s[:, 32:]) + c2[:, 32:], 0.0)

    return pl.pallas_call(
        body,
        grid=(NB,),
        in_specs=[_chunk_spec(64), _full_spec((1, 64)), _full_spec((64, 64)),
                  _full_spec((64, 64)), _full_spec((1, 64)),
                  _full_spec((1, 64)), _full_spec((1, 64))],
        out_specs=_pair_spec(32),
        out_shape=jax.ShapeDtypeStruct((2, N, 32), jnp.float32),
    )(r1, su2, gram2, W2, b2, g2, be2)


def _stats1(h1h, agg1p, eps):
    def body(h_ref, a_ref, eps_ref, p_ref, su_ref, gram_ref, s1_ref):
        i = pl.program_id(0)
        ep = 1.0 + eps_ref[0, 0]
        pa = ep * h_ref[0, :, :] + a_ref[0, :, :]
        pb = ep * h_ref[1, :, :] + a_ref[1, :, :]
        p = jnp.concatenate([pa, pb], axis=1)
        p_ref[...] = p
        _acc(su_ref, jnp.sum(p, axis=0, keepdims=True), i)
        _acc(gram_ref, _gram(p, p), i)
        s1 = jnp.concatenate(
            [jnp.sum(h_ref[0, :, :], axis=0, keepdims=True),
             jnp.sum(h_ref[1, :, :], axis=0, keepdims=True)], axis=1)
        _acc(s1_ref, s1, i)

    return pl.pallas_call(
        body,
        grid=(NB,),
        in_specs=[_pair_spec(32), _pair_spec(32), _full_spec((1, 1))],
        out_specs=(_chunk_spec(64), _full_spec((1, 64)),
                   _full_spec((64, 64)), _full_spec((1, 64))),
        out_shape=(jax.ShapeDtypeStruct((N, 64), jnp.float32),
                   jax.ShapeDtypeStruct((1, 64), jnp.float32),
                   jax.ShapeDtypeStruct((64, 64), jnp.float32),
                   jax.ShapeDtypeStruct((1, 64), jnp.float32)),
    )(h1h, agg1p, eps)


def _readout(r1, su2, gram2, W2, b2, g2, be2, sx, s1,
             P0, P1, P2, pb, M0, mb0, M1, mb1, Wm, bm, Wv, bv):
    def body(r_ref, su_ref, gram_ref, W2_ref, b2_ref, g2_ref, be2_ref,
             sx_ref, s1_ref, P0_ref, P1_ref, P2_ref, pb_ref,
             M0_ref, mb0_ref, M1_ref, mb1_ref, Wm_ref, bm_ref, Wv_ref, bv_ref,
             mean_ref, logvar_ref, s2_ref):
        i = pl.program_id(0)
        W2v = W2_ref[...]
        mw = _dot_hi(su_ref[...] / N, W2v)
        q = jnp.sum(W2v * _dot_hi(gram_ref[...] / N, W2v), axis=0,
                    keepdims=True)
        var = q - mw * mw
        a2 = g2_ref[...] * lax.rsqrt(var + 1e-5)
        c2 = be2_ref[...] - mw * a2
        h2 = jnp.maximum(_dot(r_ref[...], W2v * a2) + c2, 0.0)
        _acc(s2_ref, jnp.sum(h2, axis=0, keepdims=True), i)

        @pl.when(i == NB - 1)
        def _():
            s1 = s1_ref[...]
            score = (_dot(sx_ref[...], P0_ref[...])
                     + _dot(s1[:, :32], P1_ref[:32, :])
                     + _dot(s1[:, 32:], P1_ref[32:, :])
                     + _dot(s2_ref[...], P2_ref[...]) + pb_ref[...])
            f = jnp.maximum(_dot(score, M0_ref[...]) + mb0_ref[...], 0.0)
            f = jnp.maximum(_dot(f, M1_ref[...]) + mb1_ref[...], 0.0)
            mean_ref[...] = _dot(f, Wm_ref[...]) + bm_ref[...]
            logvar_ref[...] = _dot(f, Wv_ref[...]) + bv_ref[...]

    return pl.pallas_call(
        body,
        grid=(NB,),
        in_specs=[_chunk_spec(64), _full_spec((1, 64)), _full_spec((64, 64)),
                  _full_spec((64, 64)), _full_spec((1, 64)),
                  _full_spec((1, 64)), _full_spec((1, 64)),
                  _full_spec((1, 16)), _full_spec((1, 64)),
                  _full_spec((16, 256)), _full_spec((64, 256)),
                  _full_spec((64, 256)), _full_spec((1, 256)),
                  _full_spec((256, 256)), _full_spec((1, 256)),
                  _full_spec((256, 128)), _full_spec((1, 128)),
                  _full_spec((128, 64)), _full_spec((1, 64)),
                  _full_spec((128, 64)), _full_spec((1, 64))],
        out_specs=(_full_spec((1, 64)), _full_spec((1, 64)),
                   _full_spec((1, 64))),
        out_shape=(jax.ShapeDtypeStruct((1, 64), jnp.float32),
                   jax.ShapeDtypeStruct((1, 64), jnp.float32),
                   jax.ShapeDtypeStruct((1, 64), jnp.float32)),
    )(r1, su2, gram2, W2, b2, g2, be2, sx, s1,
      P0, P1, P2, pb, M0, mb0, M1, mb1, Wm, bm, Wv, bv)


def kernel(x, edge_index, params):
    src = edge_index[0].astype(jnp.int32)
    dst = edge_index[1].astype(jnp.int32)
    pad = R * LANES - E
    src2d = jnp.concatenate(
        [src, jnp.zeros((pad,), jnp.int32)]).reshape(R, LANES)
    trash = N + (jnp.arange(pad, dtype=jnp.int32) % (NA - N))
    dst2d = jnp.concatenate([dst, trash]).reshape(R, LANES)

    z16 = jnp.zeros((RPT, 16), jnp.float32)
    z32 = jnp.zeros((RPT, 32), jnp.float32)

    def v(name):
        return params[name].reshape(1, -1)

    eps0 = params['eps_0'].reshape(1, 1)
    eps1 = params['eps_1'].reshape(1, 1)

    agg0p = _segsum_partials_16(src2d, dst2d, x, z16)
    su, gram, sx, pooled = _stats0(x, agg0p, eps0)
    r1, su2, gram2 = _transform(pooled, su, gram, params['W1_0'], v('b1_0'),
                                v('g1_0'), v('be1_0'), 16)
    h1h = _emit_h1(r1, su2, gram2,
                   params['W2_0'], v('b2_0'), v('g_0'), v('be_0'))

    agg1p = _segsum_cols_32(src2d, dst2d, h1h[0], h1h[1], z32)
    pooled1, su1, gram1, s1 = _stats1(h1h, agg1p, eps1)
    r1b, su2b, gram2b = _transform(pooled1, su1, gram1,
                                   params['W1_1'], v('b1_1'), v('g1_1'),
                                   v('be1_1'), 64)
    pb = (params['pb_0'] + params['pb_1'] + params['pb_2']).reshape(1, -1)
    mean, logvar, _ = _readout(
        r1b, su2b, gram2b, params['W2_1'], v('b2_1'), v('g_1'), v('be_1'),
        sx, s1, params['P_0'], params['P_1'], params['P_2'], pb,
        params['M_0'], v('mb_0'), params['M_1'], v('mb_1'),
        params['Wm'], v('bm'), params['Wv'], v('bv'))
    return (mean, logvar)


# speedup vs baseline: 6.0571x; 1.0051x over previous
import functools

import jax
import jax.numpy as jnp
from jax import lax
from jax.experimental import pallas as pl
from jax.experimental.pallas import tpu as pltpu
from jax.experimental.pallas import tpu_sc as plsc

N = 50000
E = 800000
NC = 2
NS = 16
LANES = 128
R = 6400
CH = 40
K = 8
K32 = 4
NA = 50048
RPT = NA // NS

CHK = 5000
NB = N // CHK


def _segsum_partials_16(src2d, dst2d, x, zeros):
    mesh = plsc.VectorSubcoreMesh(core_axis_name="c", subcore_axis_name="s")
    rpw = R // (NC * NS)

    @functools.partial(
        pl.kernel,
        out_type=jax.ShapeDtypeStruct((NC, NA, 16), jnp.float32),
        mesh=mesh,
        compiler_params=pltpu.CompilerParams(use_tc_tiling_on_sc=False),
        scratch_types=[
            pltpu.VMEM((CH, LANES), jnp.int32),
            pltpu.VMEM((CH, LANES), jnp.int32),
            pltpu.VMEM((K, LANES, 16), jnp.float32),
            pltpu.VMEM_SHARED((NA, 16), jnp.float32),
            pltpu.SemaphoreType.DMA,
            pltpu.SemaphoreType.DMA,
        ],
    )
    def k(src_hbm, dst_hbm, x_hbm, z_hbm, out_hbm, src_v, dst_v, rows_v, acc,
          gsem, ssem):
        c = lax.axis_index("c")
        s = lax.axis_index("s")
        tbase = s * RPT
        pltpu.sync_copy(z_hbm, acc.at[pl.ds(tbase, RPT)])
        plsc.subcore_barrier()
        base = (s * NC + c) * rpw

        @pl.loop(0, rpw // CH)
        def _(g):
            pltpu.sync_copy(src_hbm.at[pl.ds(base + g * CH, CH)], src_v)
            pltpu.sync_copy(dst_hbm.at[pl.ds(base + g * CH, CH)], dst_v)

            @pl.loop(0, CH // K)
            def _(t):
                hs = [pltpu.async_copy(x_hbm.at[src_v.at[t * K + b]],
                                       rows_v.at[b], gsem)
                      for b in range(K)]
                for h in hs:
                    h.wait()
                ss = [pltpu.async_copy(rows_v.at[b],
                                       acc.at[dst_v.at[t * K + b]], ssem,
                                       add=True)
                      for b in range(K)]
                for h in ss:
                    h.wait()

        plsc.subcore_barrier()
        pltpu.sync_copy(acc.at[pl.ds(tbase, RPT)],
                        out_hbm.at[c, pl.ds(tbase, RPT)])

    return k(src2d, dst2d, x, zeros)


def _segsum_cols_32(src2d, dst2d, h0, h1, zeros):
    mesh = plsc.VectorSubcoreMesh(core_axis_name="c", subcore_axis_name="s")
    rpt_idx = R // NS

    @functools.partial(
        pl.kernel,
        out_type=jax.ShapeDtypeStruct((NC, NA, 32), jnp.float32),
        mesh=mesh,
        compiler_params=pltpu.CompilerParams(use_tc_tiling_on_sc=False),
        scratch_types=[
            pltpu.VMEM((CH, LANES), jnp.int32),
            pltpu.VMEM((CH, LANES), jnp.int32),
            pltpu.VMEM((K32, LANES, 32), jnp.float32),
            pltpu.VMEM_SHARED((NA, 32), jnp.float32),
            pltpu.SemaphoreType.DMA,
            pltpu.SemaphoreType.DMA,
        ],
    )
    def k(src_hbm, dst_hbm, h0_hbm, h1_hbm, z_hbm, out_hbm,
          src_v, dst_v, rows_v, acc, gsem, ssem):
        c = lax.axis_index("c")
        s = lax.axis_index("s")
        tbase = s * RPT
        pltpu.sync_copy(z_hbm, acc.at[pl.ds(tbase, RPT)])
        plsc.subcore_barrier()
        base = s * rpt_idx

        @pl.loop(0, rpt_idx // CH)
        def _(g):
            pltpu.sync_copy(src_hbm.at[pl.ds(base + g * CH, CH)], src_v)
            pltpu.sync_copy(dst_hbm.at[pl.ds(base + g * CH, CH)], dst_v)

            @pl.loop(0, CH // K32)
            def _(t):
                @pl.when(c == 0)
                def _():
                    hs = [pltpu.async_copy(h0_hbm.at[src_v.at[t * K32 + b]],
                                           rows_v.at[b], gsem)
                          for b in range(K32)]
                    for h in hs:
                        h.wait()

                @pl.when(c == 1)
                def _():
                    hs = [pltpu.async_copy(h1_hbm.at[src_v.at[t * K32 + b]],
                                           rows_v.at[b], gsem)
                          for b in range(K32)]
                    for h in hs:
                        h.wait()

                ss = [pltpu.async_copy(rows_v.at[b],
                                       acc.at[dst_v.at[t * K32 + b]], ssem,
                                       add=True)
                      for b in range(K32)]
                for h in ss:
                    h.wait()

        plsc.subcore_barrier()
        pltpu.sync_copy(acc.at[pl.ds(tbase, RPT)],
                        out_hbm.at[c, pl.ds(tbase, RPT)])

    return k(src2d, dst2d, h0, h1, zeros)


def _gram(a, b):
    return lax.dot_general(a, b, (((0,), (0,)), ((), ())),
                           preferred_element_type=jnp.float32,
                           precision=lax.Precision.HIGHEST)


def _dot_hi(a, b):
    return jnp.dot(a, b, preferred_element_type=jnp.float32,
                   precision=lax.Precision.HIGHEST)


def _dot(a, b):
    return jnp.dot(a, b, preferred_element_type=jnp.float32)


def _acc(ref, val, i):
    @pl.when(i == 0)
    def _():
        ref[...] = val

    @pl.when(i > 0)
    def _():
        ref[...] += val


def _chunk_spec(width):
    return pl.BlockSpec((CHK, width), lambda i: (i, 0))


def _pair_spec(width):
    return pl.BlockSpec((2, CHK, width), lambda i: (0, i, 0))


def _full_spec(shape):
    nd = len(shape)
    return pl.BlockSpec(shape, lambda i: (0,) * nd)


def _stats0(x, agg0p, eps):
    def body(x_ref, a_ref, eps_ref, su_ref, gram_ref, sx_ref, pooled_ref):
        i = pl.program_id(0)
        p = ((1.0 + eps_ref[0, 0]) * x_ref[...]
             + a_ref[0, :, :] + a_ref[1, :, :])
        pooled_ref[...] = p
        _acc(su_ref, jnp.sum(p, axis=0, keepdims=True), i)
        _acc(gram_ref, _gram(p, p), i)
        _acc(sx_ref, jnp.sum(x_ref[...], axis=0, keepdims=True), i)

    return pl.pallas_call(
        body,
        grid=(NB,),
        in_specs=[_chunk_spec(16), _pair_spec(16), _full_spec((1, 1))],
        out_specs=(_full_spec((1, 16)), _full_spec((16, 16)),
                   _full_spec((1, 16)), _chunk_spec(16)),
        out_shape=(jax.ShapeDtypeStruct((1, 16), jnp.float32),
                   jax.ShapeDtypeStruct((16, 16), jnp.float32),
                   jax.ShapeDtypeStruct((1, 16), jnp.float32),
                   jax.ShapeDtypeStruct((N, 16), jnp.float32)),
    )(x, agg0p, eps)


def _transform(pooled, su, gram, W1, b1, g1, be1, width_in):
    def body(p_ref, su_ref, gram_ref, W1_ref, b1_ref, g1_ref, be1_ref,
             r1_ref, su2_ref, gram2_ref):
        i = pl.program_id(0)
        W1v = W1_ref[...]
        mw = _dot_hi(su_ref[...] / N, W1v)
        q = jnp.sum(W1v * _dot_hi(gram_ref[...] / N, W1v), axis=0,
                    keepdims=True)
        var = q - mw * mw
        a1 = g1_ref[...] / jnp.sqrt(var + 1e-5)
        m1 = mw + b1_ref[...]
        t = _dot(p_ref[...], W1v) + b1_ref[...]
        r = jnp.maximum((t - m1) * a1 + be1_ref[...], 0.0)
        r1_ref[...] = r
        _acc(su2_ref, jnp.sum(r, axis=0, keepdims=True), i)
        _acc(gram2_ref, _gram(r, r), i)

    return pl.pallas_call(
        body,
        grid=(NB,),
        in_specs=[_chunk_spec(width_in), _full_spec((1, width_in)),
                  _full_spec((width_in, width_in)),
                  _full_spec((width_in, 64)),
                  _full_spec((1, 64)), _full_spec((1, 64)),
                  _full_spec((1, 64))],
        out_specs=(_chunk_spec(64), _full_spec((1, 64)),
                   _full_spec((64, 64))),
        out_shape=(jax.ShapeDtypeStruct((N, 64), jnp.float32),
                   jax.ShapeDtypeStruct((1, 64), jnp.float32),
                   jax.ShapeDtypeStruct((64, 64), jnp.float32)),
    )(pooled, su, gram, W1, b1, g1, be1)


def _emit_h1(r1, su2, gram2, W2, b2, g2, be2):
    def body(r_ref, su_ref, gram_ref, W2_ref, b2_ref, g2_ref, be2_ref,
             out_ref):
        W2v = W2_ref[...]
        mw = _dot_hi(su_ref[...] / N, W2v)
        q = jnp.sum(W2v * _dot_hi(gram_ref[...] / N, W2v), axis=0,
                    keepdims=True)
        var = q - mw * mw
        a2 = g2_ref[...] / jnp.sqrt(var + 1e-5)
        m2 = mw + b2_ref[...]
        t = _dot(r_ref[...], W2v) + b2_ref[...]
        h = jnp.maximum((t - m2) * a2 + be2_ref[...], 0.0)
        out_ref[0, :, :] = h[:, :32]
        out_ref[1, :, :] = h[:, 32:]

    return pl.pallas_call(
        body,
        grid=(NB,),
        in_specs=[_chunk_spec(64), _full_spec((1, 64)), _full_spec((64, 64)),
                  _full_spec((64, 64)), _full_spec((1, 64)),
                  _full_spec((1, 64)), _full_spec((1, 64))],
        out_specs=_pair_spec(32),
        out_shape=jax.ShapeDtypeStruct((2, N, 32), jnp.float32),
    )(r1, su2, gram2, W2, b2, g2, be2)


def _stats1(h1h, agg1p, eps):
    def body(h_ref, a_ref, eps_ref, p_ref, su_ref, gram_ref, s1_ref):
        i = pl.program_id(0)
        ep = 1.0 + eps_ref[0, 0]
        pa = ep * h_ref[0, :, :] + a_ref[0, :, :]
        pb = ep * h_ref[1, :, :] + a_ref[1, :, :]
        p = jnp.concatenate([pa, pb], axis=1)
        p_ref[...] = p
        _acc(su_ref, jnp.sum(p, axis=0, keepdims=True), i)
        _acc(gram_ref, _gram(p, p), i)
        s1 = jnp.concatenate(
            [jnp.sum(h_ref[0, :, :], axis=0, keepdims=True),
             jnp.sum(h_ref[1, :, :], axis=0, keepdims=True)], axis=1)
        _acc(s1_ref, s1, i)

    return pl.pallas_call(
        body,
        grid=(NB,),
        in_specs=[_pair_spec(32), _pair_spec(32), _full_spec((1, 1))],
        out_specs=(_chunk_spec(64), _full_spec((1, 64)),
                   _full_spec((64, 64)), _full_spec((1, 64))),
        out_shape=(jax.ShapeDtypeStruct((N, 64), jnp.float32),
                   jax.ShapeDtypeStruct((1, 64), jnp.float32),
                   jax.ShapeDtypeStruct((64, 64), jnp.float32),
                   jax.ShapeDtypeStruct((1, 64), jnp.float32)),
    )(h1h, agg1p, eps)


def _readout(r1, su2, gram2, W2, b2, g2, be2, sx, s1,
             P0, P1, P2, pb, M0, mb0, M1, mb1, Wm, bm, Wv, bv):
    def body(r_ref, su_ref, gram_ref, W2_ref, b2_ref, g2_ref, be2_ref,
             sx_ref, s1_ref, P0_ref, P1_ref, P2_ref, pb_ref,
             M0_ref, mb0_ref, M1_ref, mb1_ref, Wm_ref, bm_ref, Wv_ref, bv_ref,
             mean_ref, logvar_ref, s2_ref):
        i = pl.program_id(0)
        W2v = W2_ref[...]
        mw = _dot_hi(su_ref[...] / N, W2v)
        q = jnp.sum(W2v * _dot_hi(gram_ref[...] / N, W2v), axis=0,
                    keepdims=True)
        var = q - mw * mw
        a2 = g2_ref[...] / jnp.sqrt(var + 1e-5)
        m2 = mw + b2_ref[...]
        t = _dot(r_ref[...], W2v) + b2_ref[...]
        h2 = jnp.maximum((t - m2) * a2 + be2_ref[...], 0.0)
        _acc(s2_ref, jnp.sum(h2, axis=0, keepdims=True), i)

        @pl.when(i == NB - 1)
        def _():
            s1 = s1_ref[...]
            score = (_dot(sx_ref[...], P0_ref[...])
                     + _dot(s1[:, :32], P1_ref[:32, :])
                     + _dot(s1[:, 32:], P1_ref[32:, :])
                     + _dot(s2_ref[...], P2_ref[...]) + pb_ref[...])
            f = jnp.maximum(_dot(score, M0_ref[...]) + mb0_ref[...], 0.0)
            f = jnp.maximum(_dot(f, M1_ref[...]) + mb1_ref[...], 0.0)
            mean_ref[...] = _dot(f, Wm_ref[...]) + bm_ref[...]
            logvar_ref[...] = _dot(f, Wv_ref[...]) + bv_ref[...]

    return pl.pallas_call(
        body,
        grid=(NB,),
        in_specs=[_chunk_spec(64), _full_spec((1, 64)), _full_spec((64, 64)),
                  _full_spec((64, 64)), _full_spec((1, 64)),
                  _full_spec((1, 64)), _full_spec((1, 64)),
                  _full_spec((1, 16)), _full_spec((1, 64)),
                  _full_spec((16, 256)), _full_spec((64, 256)),
                  _full_spec((64, 256)), _full_spec((1, 256)),
                  _full_spec((256, 256)), _full_spec((1, 256)),
                  _full_spec((256, 128)), _full_spec((1, 128)),
                  _full_spec((128, 64)), _full_spec((1, 64)),
                  _full_spec((128, 64)), _full_spec((1, 64))],
        out_specs=(_full_spec((1, 64)), _full_spec((1, 64)),
                   _full_spec((1, 64))),
        out_shape=(jax.ShapeDtypeStruct((1, 64), jnp.float32),
                   jax.ShapeDtypeStruct((1, 64), jnp.float32),
                   jax.ShapeDtypeStruct((1, 64), jnp.float32)),
    )(r1, su2, gram2, W2, b2, g2, be2, sx, s1,
      P0, P1, P2, pb, M0, mb0, M1, mb1, Wm, bm, Wv, bv)


def kernel(x, edge_index, params):
    src = edge_index[0].astype(jnp.int32)
    dst = edge_index[1].astype(jnp.int32)
    pad = R * LANES - E
    src2d = jnp.concatenate(
        [src, jnp.zeros((pad,), jnp.int32)]).reshape(R, LANES)
    trash = N + (jnp.arange(pad, dtype=jnp.int32) % (NA - N))
    dst2d = jnp.concatenate([dst, trash]).reshape(R, LANES)

    z16 = jnp.zeros((RPT, 16), jnp.float32)
    z32 = jnp.zeros((RPT, 32), jnp.float32)

    def v(name):
        return params[name].reshape(1, -1)

    eps0 = params['eps_0'].reshape(1, 1)
    eps1 = params['eps_1'].reshape(1, 1)

    agg0p = _segsum_partials_16(src2d, dst2d, x, z16)
    su, gram, sx, pooled = _stats0(x, agg0p, eps0)
    r1, su2, gram2 = _transform(pooled, su, gram, params['W1_0'], v('b1_0'),
                                v('g1_0'), v('be1_0'), 16)
    h1h = _emit_h1(r1, su2, gram2,
                   params['W2_0'], v('b2_0'), v('g_0'), v('be_0'))

    agg1p = _segsum_cols_32(src2d, dst2d, h1h[0], h1h[1], z32)
    pooled1, su1, gram1, s1 = _stats1(h1h, agg1p, eps1)
    r1b, su2b, gram2b = _transform(pooled1, su1, gram1,
                                   params['W1_1'], v('b1_1'), v('g1_1'),
                                   v('be1_1'), 64)
    pb = (params['pb_0'] + params['pb_1'] + params['pb_2']).reshape(1, -1)
    mean, logvar, _ = _readout(
        r1b, su2b, gram2b, params['W2_1'], v('b2_1'), v('g_1'), v('be_1'),
        sx, s1, params['P_0'], params['P_1'], params['P_2'], pb,
        params['M_0'], v('mb_0'), params['M_1'], v('mb_1'),
        params['Wm'], v('bm'), params['Wv'], v('bv'))
    return (mean, logvar)


# speedup vs baseline: 6.9428x; 1.1462x over previous
import functools

import jax
import jax.numpy as jnp
from jax import lax
from jax.experimental import pallas as pl
from jax.experimental.pallas import tpu as pltpu
from jax.experimental.pallas import tpu_sc as plsc

N = 50000
E = 800000
NC = 2
NS = 16
LANES = 128
R = 6400
CH = 40
K = 8
CH32 = 20
K32 = 5
NA = 51200
RPT = NA // NS

CHK = 5000
NB = N // CHK


def _segsum_partials_16(src2d, dst2d, x, zeros):
    mesh = plsc.VectorSubcoreMesh(core_axis_name="c", subcore_axis_name="s")
    rpw = R // (NC * NS)

    @functools.partial(
        pl.kernel,
        out_type=jax.ShapeDtypeStruct((NC, NA, 16), jnp.float32),
        mesh=mesh,
        compiler_params=pltpu.CompilerParams(use_tc_tiling_on_sc=False),
        scratch_types=[
            pltpu.VMEM((CH, LANES), jnp.int32),
            pltpu.VMEM((CH, LANES), jnp.int32),
            pltpu.VMEM((K, LANES, 16), jnp.float32),
            pltpu.VMEM_SHARED((NA, 16), jnp.float32),
            pltpu.SemaphoreType.DMA,
            pltpu.SemaphoreType.DMA,
        ],
    )
    def k(src_hbm, dst_hbm, x_hbm, z_hbm, out_hbm, src_v, dst_v, rows_v, acc,
          gsem, ssem):
        c = lax.axis_index("c")
        s = lax.axis_index("s")
        tbase = s * RPT
        pltpu.sync_copy(z_hbm, acc.at[pl.ds(tbase, RPT)])
        plsc.subcore_barrier()
        base = (s * NC + c) * rpw

        @pl.loop(0, rpw // CH)
        def _(g):
            pltpu.sync_copy(src_hbm.at[pl.ds(base + g * CH, CH)], src_v)
            pltpu.sync_copy(dst_hbm.at[pl.ds(base + g * CH, CH)], dst_v)

            @pl.loop(0, CH // K)
            def _(t):
                hs = [pltpu.async_copy(x_hbm.at[src_v.at[t * K + b]],
                                       rows_v.at[b], gsem)
                      for b in range(K)]
                for h in hs:
                    h.wait()
                ss = [pltpu.async_copy(rows_v.at[b],
                                       acc.at[dst_v.at[t * K + b]], ssem,
                                       add=True)
                      for b in range(K)]
                for h in ss:
                    h.wait()

        plsc.subcore_barrier()
        pltpu.sync_copy(acc.at[pl.ds(tbase, RPT)],
                        out_hbm.at[c, pl.ds(tbase, RPT)])

    return k(src2d, dst2d, x, zeros)


def _segsum_cols_32(src2d, dst2d, h0, h1, zeros):
    mesh = plsc.VectorSubcoreMesh(core_axis_name="c", subcore_axis_name="s")
    rpt_idx = R // NS

    @functools.partial(
        pl.kernel,
        out_type=jax.ShapeDtypeStruct((NC, NA, 32), jnp.float32),
        mesh=mesh,
        compiler_params=pltpu.CompilerParams(use_tc_tiling_on_sc=False),
        scratch_types=[
            pltpu.VMEM((CH32, LANES), jnp.int32),
            pltpu.VMEM((CH32, LANES), jnp.int32),
            pltpu.VMEM((K32, LANES, 32), jnp.float32),
            pltpu.VMEM_SHARED((NA, 32), jnp.float32),
            pltpu.SemaphoreType.DMA,
            pltpu.SemaphoreType.DMA,
        ],
    )
    def k(src_hbm, dst_hbm, h0_hbm, h1_hbm, z_hbm, out_hbm,
          src_v, dst_v, rows_v, acc, gsem, ssem):
        c = lax.axis_index("c")
        s = lax.axis_index("s")
        tbase = s * RPT
        pltpu.sync_copy(z_hbm, acc.at[pl.ds(tbase, RPT)])
        plsc.subcore_barrier()
        base = s * rpt_idx

        @pl.loop(0, rpt_idx // CH32)
        def _(g):
            pltpu.sync_copy(src_hbm.at[pl.ds(base + g * CH32, CH32)], src_v)
            pltpu.sync_copy(dst_hbm.at[pl.ds(base + g * CH32, CH32)], dst_v)

            @pl.loop(0, CH32 // K32)
            def _(t):
                @pl.when(c == 0)
                def _():
                    hs = [pltpu.async_copy(h0_hbm.at[src_v.at[t * K32 + b]],
                                           rows_v.at[b], gsem)
                          for b in range(K32)]
                    for h in hs:
                        h.wait()

                @pl.when(c == 1)
                def _():
                    hs = [pltpu.async_copy(h1_hbm.at[src_v.at[t * K32 + b]],
                                           rows_v.at[b], gsem)
                          for b in range(K32)]
                    for h in hs:
                        h.wait()

                ss = [pltpu.async_copy(rows_v.at[b],
                                       acc.at[dst_v.at[t * K32 + b]], ssem,
                                       add=True)
                      for b in range(K32)]
                for h in ss:
                    h.wait()

        plsc.subcore_barrier()
        pltpu.sync_copy(acc.at[pl.ds(tbase, RPT)],
                        out_hbm.at[c, pl.ds(tbase, RPT)])

    return k(src2d, dst2d, h0, h1, zeros)


def _gram(a, b):
    return lax.dot_general(a, b, (((0,), (0,)), ((), ())),
                           preferred_element_type=jnp.float32,
                           precision=lax.Precision.HIGHEST)


def _dot_hi(a, b):
    return jnp.dot(a, b, preferred_element_type=jnp.float32,
                   precision=lax.Precision.HIGHEST)


def _dot(a, b):
    return jnp.dot(a, b, preferred_element_type=jnp.float32)


def _acc(ref, val, i):
    @pl.when(i == 0)
    def _():
        ref[...] = val

    @pl.when(i > 0)
    def _():
        ref[...] += val


def _chunk_spec(width):
    return pl.BlockSpec((CHK, width), lambda i: (i, 0))


def _pair_spec(width):
    return pl.BlockSpec((2, CHK, width), lambda i: (0, i, 0))


def _full_spec(shape):
    nd = len(shape)
    return pl.BlockSpec(shape, lambda i: (0,) * nd)


def _stats0(x, agg0p, eps):
    def body(x_ref, a_ref, eps_ref, su_ref, gram_ref, sx_ref, pooled_ref):
        i = pl.program_id(0)
        p = ((1.0 + eps_ref[0, 0]) * x_ref[...]
             + a_ref[0, :, :] + a_ref[1, :, :])
        pooled_ref[...] = p
        _acc(su_ref, jnp.sum(p, axis=0, keepdims=True), i)
        _acc(gram_ref, _gram(p, p), i)
        _acc(sx_ref, jnp.sum(x_ref[...], axis=0, keepdims=True), i)

    return pl.pallas_call(
        body,
        grid=(NB,),
        in_specs=[_chunk_spec(16), _pair_spec(16), _full_spec((1, 1))],
        out_specs=(_full_spec((1, 16)), _full_spec((16, 16)),
                   _full_spec((1, 16)), _chunk_spec(16)),
        out_shape=(jax.ShapeDtypeStruct((1, 16), jnp.float32),
                   jax.ShapeDtypeStruct((16, 16), jnp.float32),
                   jax.ShapeDtypeStruct((1, 16), jnp.float32),
                   jax.ShapeDtypeStruct((N, 16), jnp.float32)),
    )(x, agg0p, eps)


def _transform(pooled, su, gram, W1, b1, g1, be1, width_in):
    def body(p_ref, su_ref, gram_ref, W1_ref, b1_ref, g1_ref, be1_ref,
             r1_ref, su2_ref, gram2_ref):
        i = pl.program_id(0)
        W1v = W1_ref[...]
        mw = _dot_hi(su_ref[...] / N, W1v)
        q = jnp.sum(W1v * _dot_hi(gram_ref[...] / N, W1v), axis=0,
                    keepdims=True)
        var = q - mw * mw
        a1 = g1_ref[...] / jnp.sqrt(var + 1e-5)
        m1 = mw + b1_ref[...]
        t = _dot(p_ref[...], W1v) + b1_ref[...]
        r = jnp.maximum((t - m1) * a1 + be1_ref[...], 0.0)
        r1_ref[...] = r
        _acc(su2_ref, jnp.sum(r, axis=0, keepdims=True), i)
        _acc(gram2_ref, _gram(r, r), i)

    return pl.pallas_call(
        body,
        grid=(NB,),
        in_specs=[_chunk_spec(width_in), _full_spec((1, width_in)),
                  _full_spec((width_in, width_in)),
                  _full_spec((width_in, 64)),
                  _full_spec((1, 64)), _full_spec((1, 64)),
                  _full_spec((1, 64))],
        out_specs=(_chunk_spec(64), _full_spec((1, 64)),
                   _full_spec((64, 64))),
        out_shape=(jax.ShapeDtypeStruct((N, 64), jnp.float32),
                   jax.ShapeDtypeStruct((1, 64), jnp.float32),
                   jax.ShapeDtypeStruct((64, 64), jnp.float32)),
    )(pooled, su, gram, W1, b1, g1, be1)


def _emit_h1(r1, su2, gram2, W2, b2, g2, be2):
    def body(r_ref, su_ref, gram_ref, W2_ref, b2_ref, g2_ref, be2_ref,
             out_ref):
        W2v = W2_ref[...]
        mw = _dot_hi(su_ref[...] / N, W2v)
        q = jnp.sum(W2v * _dot_hi(gram_ref[...] / N, W2v), axis=0,
                    keepdims=True)
        var = q - mw * mw
        a2 = g2_ref[...] / jnp.sqrt(var + 1e-5)
        m2 = mw + b2_ref[...]
        t = _dot(r_ref[...], W2v) + b2_ref[...]
        h = jnp.maximum((t - m2) * a2 + be2_ref[...], 0.0)
        out_ref[0, :, :] = h[:, :32]
        out_ref[1, :, :] = h[:, 32:]

    return pl.pallas_call(
        body,
        grid=(NB,),
        in_specs=[_chunk_spec(64), _full_spec((1, 64)), _full_spec((64, 64)),
                  _full_spec((64, 64)), _full_spec((1, 64)),
                  _full_spec((1, 64)), _full_spec((1, 64))],
        out_specs=_pair_spec(32),
        out_shape=jax.ShapeDtypeStruct((2, N, 32), jnp.float32),
    )(r1, su2, gram2, W2, b2, g2, be2)


def _stats1(h1h, agg1p, eps):
    def body(h_ref, a_ref, eps_ref, p_ref, su_ref, gram_ref, s1_ref):
        i = pl.program_id(0)
        ep = 1.0 + eps_ref[0, 0]
        pa = ep * h_ref[0, :, :] + a_ref[0, :, :]
        pb = ep * h_ref[1, :, :] + a_ref[1, :, :]
        p = jnp.concatenate([pa, pb], axis=1)
        p_ref[...] = p
        _acc(su_ref, jnp.sum(p, axis=0, keepdims=True), i)
        _acc(gram_ref, _gram(p, p), i)
        s1 = jnp.concatenate(
            [jnp.sum(h_ref[0, :, :], axis=0, keepdims=True),
             jnp.sum(h_ref[1, :, :], axis=0, keepdims=True)], axis=1)
        _acc(s1_ref, s1, i)

    return pl.pallas_call(
        body,
        grid=(NB,),
        in_specs=[_pair_spec(32), _pair_spec(32), _full_spec((1, 1))],
        out_specs=(_chunk_spec(64), _full_spec((1, 64)),
                   _full_spec((64, 64)), _full_spec((1, 64))),
        out_shape=(jax.ShapeDtypeStruct((N, 64), jnp.float32),
                   jax.ShapeDtypeStruct((1, 64), jnp.float32),
                   jax.ShapeDtypeStruct((64, 64), jnp.float32),
                   jax.ShapeDtypeStruct((1, 64), jnp.float32)),
    )(h1h, agg1p, eps)


def _readout(r1, su2, gram2, W2, b2, g2, be2, sx, s1,
             P0, P1, P2, pb, M0, mb0, M1, mb1, Wm, bm, Wv, bv):
    def body(r_ref, su_ref, gram_ref, W2_ref, b2_ref, g2_ref, be2_ref,
             sx_ref, s1_ref, P0_ref, P1_ref, P2_ref, pb_ref,
             M0_ref, mb0_ref, M1_ref, mb1_ref, Wm_ref, bm_ref, Wv_ref, bv_ref,
             mean_ref, logvar_ref, s2_ref):
        i = pl.program_id(0)
        W2v = W2_ref[...]
        mw = _dot_hi(su_ref[...] / N, W2v)
        q = jnp.sum(W2v * _dot_hi(gram_ref[...] / N, W2v), axis=0,
                    keepdims=True)
        var = q - mw * mw
        a2 = g2_ref[...] / jnp.sqrt(var + 1e-5)
        m2 = mw + b2_ref[...]
        t = _dot(r_ref[...], W2v) + b2_ref[...]
        h2 = jnp.maximum((t - m2) * a2 + be2_ref[...], 0.0)
        _acc(s2_ref, jnp.sum(h2, axis=0, keepdims=True), i)

        @pl.when(i == NB - 1)
        def _():
            s1 = s1_ref[...]
            score = (_dot(sx_ref[...], P0_ref[...])
                     + _dot(s1[:, :32], P1_ref[:32, :])
                     + _dot(s1[:, 32:], P1_ref[32:, :])
                     + _dot(s2_ref[...], P2_ref[...]) + pb_ref[...])
            f = jnp.maximum(_dot(score, M0_ref[...]) + mb0_ref[...], 0.0)
            f = jnp.maximum(_dot(f, M1_ref[...]) + mb1_ref[...], 0.0)
            mean_ref[...] = _dot(f, Wm_ref[...]) + bm_ref[...]
            logvar_ref[...] = _dot(f, Wv_ref[...]) + bv_ref[...]

    return pl.pallas_call(
        body,
        grid=(NB,),
        in_specs=[_chunk_spec(64), _full_spec((1, 64)), _full_spec((64, 64)),
                  _full_spec((64, 64)), _full_spec((1, 64)),
                  _full_spec((1, 64)), _full_spec((1, 64)),
                  _full_spec((1, 16)), _full_spec((1, 64)),
                  _full_spec((16, 256)), _full_spec((64, 256)),
                  _full_spec((64, 256)), _full_spec((1, 256)),
                  _full_spec((256, 256)), _full_spec((1, 256)),
                  _full_spec((256, 128)), _full_spec((1, 128)),
                  _full_spec((128, 64)), _full_spec((1, 64)),
                  _full_spec((128, 64)), _full_spec((1, 64))],
        out_specs=(_full_spec((1, 64)), _full_spec((1, 64)),
                   _full_spec((1, 64))),
        out_shape=(jax.ShapeDtypeStruct((1, 64), jnp.float32),
                   jax.ShapeDtypeStruct((1, 64), jnp.float32),
                   jax.ShapeDtypeStruct((1, 64), jnp.float32)),
    )(r1, su2, gram2, W2, b2, g2, be2, sx, s1,
      P0, P1, P2, pb, M0, mb0, M1, mb1, Wm, bm, Wv, bv)


def _pad_balancing_perm():
    import numpy as np
    n_real = E // LANES
    n_pad = R - n_real
    base_real, extra = divmod(n_real, NC * NS)
    order = []
    next_real = 0
    next_pad = n_real
    for w in range(NC * NS):
        takes = base_real + (1 if w < extra else 0)
        order.extend(range(next_real, next_real + takes))
        next_real += takes
        pads = (R // (NC * NS)) - takes
        order.extend(range(next_pad, next_pad + pads))
        next_pad += pads
    return np.asarray(order, dtype=np.int32)


_PERM = _pad_balancing_perm()


def kernel(x, edge_index, params):
    src = edge_index[0].astype(jnp.int32)
    dst = edge_index[1].astype(jnp.int32)
    pad = R * LANES - E
    src2d = jnp.concatenate(
        [src, jnp.zeros((pad,), jnp.int32)]).reshape(R, LANES)[_PERM]
    trash = N + (jnp.arange(pad, dtype=jnp.int32) % (NA - N))
    dst2d = jnp.concatenate([dst, trash]).reshape(R, LANES)[_PERM]

    z16 = jnp.zeros((RPT, 16), jnp.float32)
    z32 = jnp.zeros((RPT, 32), jnp.float32)

    def v(name):
        return params[name].reshape(1, -1)

    eps0 = params['eps_0'].reshape(1, 1)
    eps1 = params['eps_1'].reshape(1, 1)

    agg0p = _segsum_partials_16(src2d, dst2d, x, z16)
    su, gram, sx, pooled = _stats0(x, agg0p, eps0)
    r1, su2, gram2 = _transform(pooled, su, gram, params['W1_0'], v('b1_0'),
                                v('g1_0'), v('be1_0'), 16)
    h1h = _emit_h1(r1, su2, gram2,
                   params['W2_0'], v('b2_0'), v('g_0'), v('be_0'))

    agg1p = _segsum_cols_32(src2d, dst2d, h1h[0], h1h[1], z32)
    pooled1, su1, gram1, s1 = _stats1(h1h, agg1p, eps1)
    r1b, su2b, gram2b = _transform(pooled1, su1, gram1,
                                   params['W1_1'], v('b1_1'), v('g1_1'),
                                   v('be1_1'), 64)
    pb = (params['pb_0'] + params['pb_1'] + params['pb_2']).reshape(1, -1)
    mean, logvar, _ = _readout(
        r1b, su2b, gram2b, params['W2_1'], v('b2_1'), v('g_1'), v('be_1'),
        sx, s1, params['P_0'], params['P_1'], params['P_2'], pb,
        params['M_0'], v('mb_0'), params['M_1'], v('mb_1'),
        params['Wm'], v('bm'), params['Wv'], v('bv'))
    return (mean, logvar)
